# baseline (speedup 1.0000x reference)
import jax
import jax.numpy as jnp
from jax.experimental import pallas as pl
from jax.experimental.pallas import tpu as pltpu

_C = 10
_CP = 16
_B = 1024
_D = 4096
_BLK = 256
_NB = _B // _BLK
_GAMMA = 0.1
_EPS = 1e-8


def _body(fs_ref, ys_ref, ft_ref, out_ref,
          acc_ref, proto_ref, npr_ref):
    i = pl.program_id(0)

    @pl.when(i == 0)
    def _init():
        acc_ref[...] = jnp.zeros_like(acc_ref)

    @pl.when(i < _NB)
    def _accum():
        ys = ys_ref[0, 0, :]
        classes = jax.lax.broadcasted_iota(jnp.int32, (_BLK, _CP), 1)
        onehot = (ys[:, None] == classes).astype(jnp.bfloat16)
        fs = fs_ref[...]
        mask = jnp.uint32(0xFFFF0000)
        bits = jax.lax.bitcast_convert_type(fs, jnp.uint32)
        hi_f = jax.lax.bitcast_convert_type(bits & mask, jnp.float32)
        r1 = fs - hi_f
        r1b = jax.lax.bitcast_convert_type(r1, jnp.uint32)
        mid_f = jax.lax.bitcast_convert_type(r1b & mask, jnp.float32)
        r2 = r1 - mid_f
        fs3 = jnp.concatenate(
            [hi_f.astype(jnp.bfloat16),
             mid_f.astype(jnp.bfloat16),
             r2.astype(jnp.bfloat16)], axis=0)
        onehot3 = jnp.concatenate([onehot, onehot, onehot], axis=0)
        contrib = jax.lax.dot_general(
            onehot3, fs3, (((0,), (0,)), ((), ())),
            preferred_element_type=jnp.float32)
        ones = jnp.ones((_BLK, 128), jnp.bfloat16)
        cnt_contrib = jax.lax.dot_general(
            onehot, ones, (((0,), (0,)), ((), ())),
            preferred_element_type=jnp.float32)
        acc_ref[:, :_D] += contrib
        acc_ref[:, _D:] += cnt_contrib

    @pl.when(i == _NB)
    def _finalize():
        sums = acc_ref[:, :_D]
        counts = acc_ref[:, _D:_D + 1]
        proto_new = jnp.where(
            counts > 0.0, sums / jnp.maximum(counts, 1.0), 0.0)
        proto = (1.0 - _GAMMA) * proto_new
        proto_ref[...] = proto.astype(jnp.bfloat16)
        npr_ref[...] = jnp.sqrt(
            jnp.sum(proto * proto, axis=1)).reshape(1, _CP)

    @pl.when(i >= _NB)
    def _assign():
        ft = ft_ref[...]
        raw = jax.lax.dot_general(
            ft.astype(jnp.bfloat16), proto_ref[...],
            (((1,), (1,)), ((), ())),
            preferred_element_type=jnp.float32)
        nf = jnp.sqrt(jnp.sum(ft * ft, axis=1, keepdims=True))
        cos = raw / jnp.maximum(nf * npr_ref[...], _EPS)
        lane = jax.lax.broadcasted_iota(jnp.int32, (_BLK, _CP), 1)
        cos = jnp.where(lane < _C, cos, jnp.inf)
        labels = jnp.argmin(cos, axis=1).astype(jnp.int32)
        out_ref[...] = labels.reshape(1, 1, _BLK)


def kernel(fs, ys, ft):
    ys3 = ys.astype(jnp.int32).reshape(_NB, 1, _BLK)
    out = pl.pallas_call(
        _body,
        grid=(2 * _NB,),
        in_specs=[
            pl.BlockSpec((_BLK, _D), lambda i: (jnp.minimum(i, _NB - 1), 0)),
            pl.BlockSpec((1, 1, _BLK), lambda i: (jnp.minimum(i, _NB - 1), 0, 0)),
            pl.BlockSpec((_BLK, _D), lambda i: (jnp.maximum(i - _NB, 0), 0)),
        ],
        out_specs=pl.BlockSpec((1, 1, _BLK), lambda i: (jnp.maximum(i - _NB, 0), 0, 0)),
        out_shape=jax.ShapeDtypeStruct((_NB, 1, _BLK), jnp.int32),
        scratch_shapes=[
            pltpu.VMEM((_CP, _D + 128), jnp.float32),
            pltpu.VMEM((_CP, _D), jnp.bfloat16),
            pltpu.VMEM((1, _CP), jnp.float32),
        ],
        compiler_params=pltpu.CompilerParams(
            dimension_semantics=("arbitrary",)),
    )(fs, ys3, ft)
    return out.reshape(_B)


# speedup vs baseline: 1.1665x; 1.1665x over previous
import jax
import jax.numpy as jnp
from jax.experimental import pallas as pl
from jax.experimental.pallas import tpu as pltpu

_C = 10
_CP = 16
_B = 1024
_D = 4096
_BLK = 256
_NB = _B // _BLK
_GAMMA = 0.1
_EPS = 1e-8


def _body(fs_ref, ys_ref, ft_hbm_ref, out_ref,
          acc_ref, ftv_ref, proto_ref, npr_ref, sems):
    i = pl.program_id(0)

    @pl.when(i == 0)
    def _init():
        acc_ref[...] = jnp.zeros_like(acc_ref)

    @pl.when(i < _NB)
    def _accum():
        pltpu.make_async_copy(
            ft_hbm_ref.at[pl.ds(i * _BLK, _BLK), :],
            ftv_ref.at[pl.ds(i * _BLK, _BLK), :],
            sems.at[i]).start()

        ys = ys_ref[0, 0, :]
        classes = jax.lax.broadcasted_iota(jnp.int32, (_BLK, _CP), 1)
        onehot = (ys[:, None] == classes).astype(jnp.bfloat16)
        fs = fs_ref[...]
        mask = jnp.uint32(0xFFFF0000)
        bits = jax.lax.bitcast_convert_type(fs, jnp.uint32)
        hi_f = jax.lax.bitcast_convert_type(bits & mask, jnp.float32)
        r1 = fs - hi_f
        r1b = jax.lax.bitcast_convert_type(r1, jnp.uint32)
        mid_f = jax.lax.bitcast_convert_type(r1b & mask, jnp.float32)
        r2 = r1 - mid_f
        fs_hi = hi_f.astype(jnp.bfloat16)
        fs_mid = mid_f.astype(jnp.bfloat16)
        fs_lo = r2.astype(jnp.bfloat16)
        dims = (((0,), (0,)), ((), ()))
        contrib = (jax.lax.dot_general(
                       onehot, fs_hi, dims,
                       preferred_element_type=jnp.float32)
                   + jax.lax.dot_general(
                       onehot, fs_mid, dims,
                       preferred_element_type=jnp.float32)
                   + jax.lax.dot_general(
                       onehot, fs_lo, dims,
                       preferred_element_type=jnp.float32))
        ones = jnp.ones((_BLK, 128), jnp.bfloat16)
        cnt_contrib = jax.lax.dot_general(
            onehot, ones, (((0,), (0,)), ((), ())),
            preferred_element_type=jnp.float32)
        acc_ref[:, :_D] += contrib
        acc_ref[:, _D:] += cnt_contrib

    @pl.when(i == _NB)
    def _finalize():
        sums = acc_ref[:, :_D]
        counts = acc_ref[:, _D:_D + 1]
        proto_new = jnp.where(
            counts > 0.0, sums / jnp.maximum(counts, 1.0), 0.0)
        proto = (1.0 - _GAMMA) * proto_new
        proto_ref[...] = proto.astype(jnp.bfloat16)
        npr_ref[...] = jnp.sqrt(
            jnp.sum(proto * proto, axis=1)).reshape(1, _CP)

    @pl.when(i >= _NB)
    def _assign():
        j = i - _NB
        pltpu.make_async_copy(
            ft_hbm_ref.at[pl.ds(j * _BLK, _BLK), :],
            ftv_ref.at[pl.ds(j * _BLK, _BLK), :],
            sems.at[j]).wait()
        ft = ftv_ref[pl.ds(j * _BLK, _BLK), :]
        raw = jax.lax.dot_general(
            ft.astype(jnp.bfloat16), proto_ref[...],
            (((1,), (1,)), ((), ())),
            preferred_element_type=jnp.float32)
        nf = jnp.sqrt(jnp.sum(ft * ft, axis=1, keepdims=True))
        cos = raw / jnp.maximum(nf * npr_ref[...], _EPS)
        lane = jax.lax.broadcasted_iota(jnp.int32, (_BLK, _CP), 1)
        cos = jnp.where(lane < _C, cos, jnp.inf)
        labels = jnp.argmin(cos, axis=1).astype(jnp.int32)
        out_ref[...] = labels.reshape(1, 1, _BLK)


def kernel(fs, ys, ft):
    ys3 = ys.astype(jnp.int32).reshape(_NB, 1, _BLK)
    out = pl.pallas_call(
        _body,
        grid=(2 * _NB,),
        in_specs=[
            pl.BlockSpec((_BLK, _D), lambda i: (jnp.minimum(i, _NB - 1), 0)),
            pl.BlockSpec((1, 1, _BLK), lambda i: (jnp.minimum(i, _NB - 1), 0, 0)),
            pl.BlockSpec(memory_space=pltpu.MemorySpace.HBM),
        ],
        out_specs=pl.BlockSpec((1, 1, _BLK), lambda i: (jnp.maximum(i - _NB, 0), 0, 0)),
        out_shape=jax.ShapeDtypeStruct((_NB, 1, _BLK), jnp.int32),
        scratch_shapes=[
            pltpu.VMEM((_CP, _D + 128), jnp.float32),
            pltpu.VMEM((_B, _D), jnp.float32),
            pltpu.VMEM((_CP, _D), jnp.bfloat16),
            pltpu.VMEM((1, _CP), jnp.float32),
            pltpu.SemaphoreType.DMA((_NB,)),
        ],
        compiler_params=pltpu.CompilerParams(
            dimension_semantics=("arbitrary",)),
    )(fs, ys3, ft)
    return out.reshape(_B)


# speedup vs baseline: 1.2289x; 1.0535x over previous
import jax
import jax.numpy as jnp
from jax.experimental import pallas as pl
from jax.experimental.pallas import tpu as pltpu

_C = 10
_CP = 16
_B = 1024
_D = 4096
_BLK = 512
_NB = _B // _BLK
_GAMMA = 0.1
_EPS = 1e-8


def _body(fs_ref, ys_ref, ft_hbm_ref, out_ref,
          acc_ref, ftv_ref, proto_ref, npr_ref, sems):
    i = pl.program_id(0)

    @pl.when(i == 0)
    def _init():
        acc_ref[...] = jnp.zeros_like(acc_ref)

    @pl.when(i < _NB)
    def _accum():
        pltpu.make_async_copy(
            ft_hbm_ref.at[pl.ds(i * _BLK, _BLK), :],
            ftv_ref.at[pl.ds(i * _BLK, _BLK), :],
            sems.at[i]).start()

        ys = ys_ref[0, 0, :]
        classes = jax.lax.broadcasted_iota(jnp.int32, (_BLK, _CP), 1)
        onehot = (ys[:, None] == classes).astype(jnp.bfloat16)
        fs = fs_ref[...]
        mask = jnp.uint32(0xFFFF0000)
        bits = jax.lax.bitcast_convert_type(fs, jnp.uint32)
        hi_f = jax.lax.bitcast_convert_type(bits & mask, jnp.float32)
        r1 = fs - hi_f
        r1b = jax.lax.bitcast_convert_type(r1, jnp.uint32)
        mid_f = jax.lax.bitcast_convert_type(r1b & mask, jnp.float32)
        r2 = r1 - mid_f
        fs_hi = hi_f.astype(jnp.bfloat16)
        fs_mid = mid_f.astype(jnp.bfloat16)
        fs_lo = r2.astype(jnp.bfloat16)
        dims = (((0,), (0,)), ((), ()))
        contrib = (jax.lax.dot_general(
                       onehot, fs_hi, dims,
                       preferred_element_type=jnp.float32)
                   + jax.lax.dot_general(
                       onehot, fs_mid, dims,
                       preferred_element_type=jnp.float32)
                   + jax.lax.dot_general(
                       onehot, fs_lo, dims,
                       preferred_element_type=jnp.float32))
        ones = jnp.ones((_BLK, 128), jnp.bfloat16)
        cnt_contrib = jax.lax.dot_general(
            onehot, ones, (((0,), (0,)), ((), ())),
            preferred_element_type=jnp.float32)
        acc_ref[:, :_D] += contrib
        acc_ref[:, _D:] += cnt_contrib

    @pl.when(i == _NB)
    def _finalize():
        sums = acc_ref[:, :_D]
        counts = acc_ref[:, _D:_D + 1]
        proto_new = jnp.where(
            counts > 0.0, sums / jnp.maximum(counts, 1.0), 0.0)
        proto = (1.0 - _GAMMA) * proto_new
        proto_ref[...] = proto.astype(jnp.bfloat16)
        npr_ref[...] = jnp.sqrt(
            jnp.sum(proto * proto, axis=1)).reshape(1, _CP)

    @pl.when(i >= _NB)
    def _assign():
        j = i - _NB
        pltpu.make_async_copy(
            ft_hbm_ref.at[pl.ds(j * _BLK, _BLK), :],
            ftv_ref.at[pl.ds(j * _BLK, _BLK), :],
            sems.at[j]).wait()
        ft = ftv_ref[pl.ds(j * _BLK, _BLK), :]
        raw = jax.lax.dot_general(
            ft.astype(jnp.bfloat16), proto_ref[...],
            (((1,), (1,)), ((), ())),
            preferred_element_type=jnp.float32)
        nf = jnp.sqrt(jnp.sum(ft * ft, axis=1, keepdims=True))
        cos = raw / jnp.maximum(nf * npr_ref[...], _EPS)
        lane = jax.lax.broadcasted_iota(jnp.int32, (_BLK, _CP), 1)
        cos = jnp.where(lane < _C, cos, jnp.inf)
        labels = jnp.argmin(cos, axis=1).astype(jnp.int32)
        out_ref[...] = labels.reshape(1, 1, _BLK)


def kernel(fs, ys, ft):
    ys3 = ys.astype(jnp.int32).reshape(_NB, 1, _BLK)
    out = pl.pallas_call(
        _body,
        grid=(2 * _NB,),
        in_specs=[
            pl.BlockSpec((_BLK, _D), lambda i: (jnp.minimum(i, _NB - 1), 0)),
            pl.BlockSpec((1, 1, _BLK), lambda i: (jnp.minimum(i, _NB - 1), 0, 0)),
            pl.BlockSpec(memory_space=pltpu.MemorySpace.HBM),
        ],
        out_specs=pl.BlockSpec((1, 1, _BLK), lambda i: (jnp.maximum(i - _NB, 0), 0, 0)),
        out_shape=jax.ShapeDtypeStruct((_NB, 1, _BLK), jnp.int32),
        scratch_shapes=[
            pltpu.VMEM((_CP, _D + 128), jnp.float32),
            pltpu.VMEM((_B, _D), jnp.float32),
            pltpu.VMEM((_CP, _D), jnp.bfloat16),
            pltpu.VMEM((1, _CP), jnp.float32),
            pltpu.SemaphoreType.DMA((_NB,)),
        ],
        compiler_params=pltpu.CompilerParams(
            dimension_semantics=("arbitrary",)),
    )(fs, ys3, ft)
    return out.reshape(_B)
